# baseline (speedup 1.0000x reference)
import jax
import jax.numpy as jnp
from jax import lax
from jax.experimental import pallas as pl
from jax.experimental.pallas import tpu as pltpu
from jax.experimental.pallas import tpu_sc as plsc

_ROWS = 16384
_D = 2048
_NC = 2
_NS = 16
_NW = _NC * _NS
_ROWS_PER_W = _ROWS // _NW
_CHUNK = 16
_NCHUNK = _ROWS_PER_W // _CHUNK


def _sc_copy_body(x_hbm, o_hbm, buf0, buf1, r0, r1, w0, w1):
    wid = lax.axis_index("s") * _NC + lax.axis_index("c")
    base = wid * _ROWS_PER_W
    bufs = (buf0, buf1)
    rsems = (r0, r1)
    wsems = (w0, w1)
    reads = [None, None]
    writes = [None, None]
    for g in range(_NCHUNK):
        b = g & 1
        if g == 0:
            reads[b] = pltpu.async_copy(
                x_hbm.at[pl.ds(base, _CHUNK)], bufs[b], rsems[b])
        reads[b].wait()
        writes[b] = pltpu.async_copy(
            bufs[b], o_hbm.at[pl.ds(base + g * _CHUNK, _CHUNK)], wsems[b])
        if g + 1 < _NCHUNK:
            nb = 1 - b
            if writes[nb] is not None:
                writes[nb].wait()
            reads[nb] = pltpu.async_copy(
                x_hbm.at[pl.ds(base + (g + 1) * _CHUNK, _CHUNK)],
                bufs[nb], rsems[nb])
    writes[0].wait()
    writes[1].wait()


def kernel(x, rel_pos_bias):
    del rel_pos_bias
    b, s, d = x.shape
    x2 = x.reshape(b * s, d)
    mesh = plsc.VectorSubcoreMesh(core_axis_name="c", subcore_axis_name="s")
    sc_copy = pl.kernel(
        _sc_copy_body,
        out_type=jax.ShapeDtypeStruct((b * s, d), x.dtype),
        mesh=mesh,
        scratch_types=[
            pltpu.VMEM((_CHUNK, _D), jnp.float32),
            pltpu.VMEM((_CHUNK, _D), jnp.float32),
            pltpu.SemaphoreType.DMA,
            pltpu.SemaphoreType.DMA,
            pltpu.SemaphoreType.DMA,
            pltpu.SemaphoreType.DMA,
        ],
    )
    return sc_copy(x2).reshape(b, s, d)


# speedup vs baseline: 1.0125x; 1.0125x over previous
import jax
import jax.numpy as jnp
from jax import lax
from jax.experimental import pallas as pl
from jax.experimental.pallas import tpu as pltpu
from jax.experimental.pallas import tpu_sc as plsc

_ROWS = 16384
_D = 2048
_NC = 2
_NS = 16
_NW = _NC * _NS
_ROWS_PER_W = _ROWS // _NW
_CHUNK = 16
_NCHUNK = _ROWS_PER_W // _CHUNK
_NBUF = 3


def _sc_copy_body(x_hbm, o_hbm, *rest):
    bufs = rest[:_NBUF]
    rsems = rest[_NBUF:2 * _NBUF]
    wsems = rest[2 * _NBUF:3 * _NBUF]
    wid = lax.axis_index("s") * _NC + lax.axis_index("c")
    base = wid * _ROWS_PER_W
    reads = [None] * _NBUF
    writes = [None] * _NBUF
    for g in range(_NBUF - 1):
        reads[g] = pltpu.async_copy(
            x_hbm.at[pl.ds(base + g * _CHUNK, _CHUNK)], bufs[g], rsems[g])
    for g in range(_NCHUNK):
        b = g % _NBUF
        reads[b].wait()
        writes[b] = pltpu.async_copy(
            bufs[b], o_hbm.at[pl.ds(base + g * _CHUNK, _CHUNK)], wsems[b])
        nxt = g + _NBUF - 1
        if nxt < _NCHUNK:
            nb = nxt % _NBUF
            if writes[nb] is not None:
                writes[nb].wait()
            reads[nb] = pltpu.async_copy(
                x_hbm.at[pl.ds(base + nxt * _CHUNK, _CHUNK)],
                bufs[nb], rsems[nb])
    for b in range(_NBUF):
        if writes[b] is not None:
            writes[b].wait()


def kernel(x, rel_pos_bias):
    del rel_pos_bias
    b, s, d = x.shape
    x2 = x.reshape(b * s, d)
    mesh = plsc.VectorSubcoreMesh(core_axis_name="c", subcore_axis_name="s")
    sc_copy = pl.kernel(
        _sc_copy_body,
        out_type=jax.ShapeDtypeStruct((b * s, d), x.dtype),
        mesh=mesh,
        scratch_types=(
            [pltpu.VMEM((_CHUNK, _D), jnp.float32)] * _NBUF
            + [pltpu.SemaphoreType.DMA] * (2 * _NBUF)
        ),
    )
    return sc_copy(x2).reshape(b, s, d)


# speedup vs baseline: 1.3970x; 1.3797x over previous
import jax
import jax.numpy as jnp
from jax.experimental import pallas as pl
from jax.experimental.pallas import tpu as pltpu

_ROWS = 16384
_D = 2048
_CHUNK = 1024
_NCHUNK = _ROWS // _CHUNK
_NBUF = 3


def _copy_body(x_ref, o_ref, *rest):
    bufs = rest[:_NBUF]
    rsems = rest[_NBUF:2 * _NBUF]
    wsems = rest[2 * _NBUF:3 * _NBUF]
    reads = [None] * _NBUF
    writes = [None] * _NBUF
    for g in range(_NBUF - 1):
        reads[g] = pltpu.make_async_copy(
            x_ref.at[pl.ds(g * _CHUNK, _CHUNK)], bufs[g], rsems[g])
        reads[g].start()
    for g in range(_NCHUNK):
        b = g % _NBUF
        reads[b].wait()
        writes[b] = pltpu.make_async_copy(
            bufs[b], o_ref.at[pl.ds(g * _CHUNK, _CHUNK)], wsems[b])
        writes[b].start()
        nxt = g + _NBUF - 1
        if nxt < _NCHUNK:
            nb = nxt % _NBUF
            if writes[nb] is not None:
                writes[nb].wait()
            reads[nb] = pltpu.make_async_copy(
                x_ref.at[pl.ds(nxt * _CHUNK, _CHUNK)], bufs[nb], rsems[nb])
            reads[nb].start()
    for b in range(_NBUF):
        if writes[b] is not None:
            writes[b].wait()


def kernel(x, rel_pos_bias):
    del rel_pos_bias
    b, s, d = x.shape
    x2 = x.reshape(b * s, d)
    out = pl.pallas_call(
        _copy_body,
        out_shape=jax.ShapeDtypeStruct((b * s, d), x.dtype),
        in_specs=[pl.BlockSpec(memory_space=pl.ANY)],
        out_specs=pl.BlockSpec(memory_space=pl.ANY),
        scratch_shapes=(
            [pltpu.VMEM((_CHUNK, _D), jnp.float32)] * _NBUF
            + [pltpu.SemaphoreType.DMA] * (2 * _NBUF)
        ),
    )(x2)
    return out.reshape(b, s, d)
